# baseline (speedup 1.0000x reference)
import functools

import jax
import jax.numpy as jnp
from jax import lax
from jax.experimental import pallas as pl
from jax.experimental.pallas import tpu as pltpu
from jax.experimental.pallas import tpu_sc as plsc

N_USERS = 50000
N_ITEMS = 50000
NN = N_USERS + N_ITEMS
EMB = 32
HALF = 16
E = 1600000
BATCH = 4096
REG = 1e-05

NC = 2
NS = 16

BE = 512
CHUNK = 128
NCH = BE // CHUNK
NBLK = 196

NNP = 100352
ROWS_PER_TILE = NNP // NS
ZCH = 784


def _spmm_body(ego_ref, colslo_ref, colshi_ref, rows2_ref, vals_ref,
               out_ref,
               idx_0, vals_0, rows_0, idx_1, vals_1, rows_1,
               idx_2, vals_2, rows_2, idx_3, vals_3, rows_3,
               gbuf_a, gbuf_b, acc, *sems):
  stsem = sems[:4]
  gsem = sems[4:12]
  ssem = sems[12:20]
  c = lax.axis_index("c")
  s = lax.axis_index("s")
  sets = ((idx_0, vals_0, rows_0), (idx_1, vals_1, rows_1),
          (idx_2, vals_2, rows_2), (idx_3, vals_3, rows_3))
  gbufs = (gbuf_a, gbuf_b)

  @pl.loop(0, ZCH)
  def _(r):
    gbuf_a[r] = jnp.zeros((HALF,), jnp.float32)

  for t in range(ROWS_PER_TILE // ZCH):
    pltpu.sync_copy(gbuf_a.at[pl.ds(0, ZCH)],
                    acc.at[pl.ds(s * ROWS_PER_TILE + t * ZCH, ZCH)])
  plsc.subcore_barrier()

  def blk_off(b):
    fake = jnp.logical_and(b == NBLK - 1, s >= 5)
    return jnp.where(fake, 0, (s + 16 * b) * BE), fake

  def stage_fire(b, m):
    st_idx, st_vals, st_rows = sets[m]
    off, _ = blk_off(b)

    @pl.when(c == 0)
    def _():
      pltpu.async_copy(colslo_ref.at[pl.ds(off, BE)], st_idx, stsem[m])

    @pl.when(c == 1)
    def _():
      pltpu.async_copy(colshi_ref.at[pl.ds(off, BE)], st_idx, stsem[m])

    pltpu.async_copy(vals_ref.at[pl.ds(off, BE)], st_vals, stsem[m])
    pltpu.async_copy(rows2_ref.at[pl.ds(off // CHUNK, NCH), :], st_rows,
                     stsem[m])

  def stage_wait(m):
    st_idx, st_vals, st_rows = sets[m]
    pltpu.make_async_copy(colslo_ref.at[pl.ds(0, BE)], st_idx,
                          stsem[m]).wait()
    pltpu.make_async_copy(vals_ref.at[pl.ds(0, BE)], st_vals,
                          stsem[m]).wait()
    pltpu.make_async_copy(rows2_ref.at[pl.ds(0, NCH), :], st_rows,
                          stsem[m]).wait()

  def fire_gathers(m, p):
    st_idx = sets[m][0]
    gb = gbufs[p]
    for j in range(NCH):
      pltpu.async_copy(ego_ref.at[st_idx.at[pl.ds(j * CHUNK, CHUNK)]],
                       gb.at[pl.ds(j * CHUNK, CHUNK)], gsem[p * NCH + j])

  def scale_chunk(gb, j, st_vals):
    @plsc.parallel_loop(0, CHUNK // 16, unroll=4)
    def _(g):
      base_r = j * CHUNK + g * 16
      vv = st_vals[pl.ds(base_r, 16)]
      for k in range(16):
        r = base_r + k
        gb[r] = gb[r] * vv[k]

  def drain_scatters(p, mrows):
    rws = sets[mrows][2]
    gb = gbufs[p]
    for j in range(NCH):
      pltpu.make_async_copy(gb.at[pl.ds(j * CHUNK, CHUNK)],
                            acc.at[rws.at[j]], ssem[p * NCH + j]).wait()

  def proc(t, k):
    b = 4 * t + k
    p = k % 2
    q = 1 - p
    gb = gbufs[p]
    mc = k % 4
    mn = (k + 1) % 4
    mr = (k + 3) % 4
    _, fake = blk_off(b)

    if k == 0:
      @pl.when(t > 0)
      def _():
        drain_scatters(q, mr)
    else:
      drain_scatters(q, mr)

    if k == 0:
      stage_fire(b + 3, mr)
    else:
      @pl.when(t < NBLK // 4 - 1)
      def _():
        stage_fire(b + 3, mr)

    def prefetch():
      stage_wait(mn)
      fire_gathers(mn, q)

    if k == 3:
      @pl.when(t < NBLK // 4 - 1)
      def _():
        prefetch()
    else:
      prefetch()

    @pl.when(fake)
    def _():
      @pl.loop(0, BE // 16)
      def _(z):
        sets[mc][1][pl.ds(z * 16, 16)] = jnp.zeros((16,), jnp.float32)

    for j in range(NCH):
      pltpu.make_async_copy(ego_ref.at[sets[mc][0].at[pl.ds(j * CHUNK,
                                                            CHUNK)]],
                            gb.at[pl.ds(j * CHUNK, CHUNK)],
                            gsem[p * NCH + j]).wait()
      scale_chunk(gb, j, sets[mc][1])
      pltpu.async_copy(gb.at[pl.ds(j * CHUNK, CHUNK)],
                       acc.at[sets[mc][2].at[j]], ssem[p * NCH + j],
                       add=True)

  stage_fire(0, 0)
  stage_fire(1, 1)
  stage_fire(2, 2)
  stage_wait(0)
  fire_gathers(0, 0)

  @pl.loop(0, NBLK // 4)
  def _(t):
    for k in range(4):
      proc(t, k)

  drain_scatters((NBLK - 1) % 2, (NBLK - 1) % 4)

  plsc.subcore_barrier()

  for t in range(ROWS_PER_TILE // ZCH):
    r0 = s * ROWS_PER_TILE + t * ZCH
    pltpu.sync_copy(acc.at[pl.ds(r0, ZCH)], gbuf_a.at[pl.ds(0, ZCH)])
    pltpu.sync_copy(gbuf_a.at[pl.ds(0, ZCH)], out_ref.at[c, pl.ds(r0, ZCH), :])


_spmm_call = functools.partial(
    pl.kernel,
    out_type=jax.ShapeDtypeStruct((NC, NNP, HALF), jnp.float32),
    mesh=plsc.VectorSubcoreMesh(core_axis_name="c", subcore_axis_name="s",
                                num_cores=NC, num_subcores=NS),
    scratch_types=[
        pltpu.VMEM((BE,), jnp.int32),
        pltpu.VMEM((BE,), jnp.float32),
        pltpu.VMEM((NCH, CHUNK), jnp.int32),
    ] * 4 + [
        pltpu.VMEM((BE, HALF), jnp.float32),
        pltpu.VMEM((BE, HALF), jnp.float32),
        pltpu.VMEM_SHARED((NNP, HALF), jnp.float32),
    ] + [pltpu.SemaphoreType.DMA] * 20,
    compiler_params=pltpu.CompilerParams(use_tc_tiling_on_sc=False),
)(_spmm_body)


def _dense_body(lohi_ref, ego_ref, w1_ref, b1_ref, w2_ref, b2_ref,
                plo_ref, phi_ref, ones_ref, exp_ref, e_ref, tnorm_ref):
  side = (jnp.dot(lohi_ref[0], plo_ref[...],
                  preferred_element_type=jnp.float32) +
          jnp.dot(lohi_ref[1], phi_ref[...],
                  preferred_element_type=jnp.float32))
  ego = ego_ref[...]
  side_li = side + ego
  simple = jnp.dot(side_li, w1_ref[...], preferred_element_type=jnp.float32)
  inter = jnp.dot(ego * side, w2_ref[...], preferred_element_type=jnp.float32)
  e = simple + inter + b1_ref[...] + b2_ref[...]
  e = jnp.where(e >= 0, e, 0.01 * e)
  e_ref[...] = e
  nrm2 = jnp.dot(e * e, ones_ref[...], preferred_element_type=jnp.float32)
  scale = 1.0 / jnp.maximum(jnp.sqrt(nrm2), 1e-12)
  sw = jnp.dot(scale, exp_ref[...], preferred_element_type=jnp.float32)
  tnorm_ref[...] = e * sw


NP8 = NNP // 8
RB8 = NP8 // 4

_dense_call = pl.pallas_call(
    _dense_body,
    grid=(NP8 // RB8,),
    in_specs=[
        pl.BlockSpec((NC, RB8, 128), lambda i: (0, i, 0)),
        pl.BlockSpec((RB8, 256), lambda i: (i, 0)),
        pl.BlockSpec((256, 256), lambda i: (0, 0)),
        pl.BlockSpec((1, 256), lambda i: (0, 0)),
        pl.BlockSpec((256, 256), lambda i: (0, 0)),
        pl.BlockSpec((1, 256), lambda i: (0, 0)),
        pl.BlockSpec((128, 256), lambda i: (0, 0)),
        pl.BlockSpec((128, 256), lambda i: (0, 0)),
        pl.BlockSpec((256, 8), lambda i: (0, 0)),
        pl.BlockSpec((8, 256), lambda i: (0, 0)),
    ],
    out_specs=[
        pl.BlockSpec((RB8, 256), lambda i: (i, 0)),
        pl.BlockSpec((RB8, 256), lambda i: (i, 0)),
    ],
    out_shape=[
        jax.ShapeDtypeStruct((NP8, 256), jnp.float32),
        jax.ShapeDtypeStruct((NP8, 256), jnp.float32),
    ],
)

BPT = BATCH // (NC * NS)


def _gather_body(tu_ref, ti_ref, t1_ref, t2_ref, t3_ref,
                 ui_ref, pi_ref, ni_ref, i0_ref, j0_ref,
                 uo_ref, po_ref, no_ref, iu, i0, buf, sem):
  c = lax.axis_index("c")
  s = lax.axis_index("s")
  w = s * NC + c
  base = w * BPT
  for t0_ref, idx_ref, idx0_ref, out_ref in (
      (tu_ref, ui_ref, ui_ref, uo_ref),
      (ti_ref, pi_ref, i0_ref, po_ref),
      (ti_ref, ni_ref, j0_ref, no_ref)):
    pltpu.sync_copy(idx_ref.at[pl.ds(base, BPT)], iu)
    pltpu.sync_copy(idx0_ref.at[pl.ds(base, BPT)], i0)
    pltpu.async_copy(t0_ref.at[i0], buf, sem).wait()
    pltpu.sync_copy(buf, out_ref.at[pl.ds(base, BPT), pl.ds(0, EMB)])
    for k, tk in ((1, t1_ref), (2, t2_ref), (3, t3_ref)):
      pltpu.async_copy(tk.at[iu], buf, sem).wait()
      pltpu.sync_copy(buf, out_ref.at[pl.ds(base, BPT),
                                      pl.ds(k * EMB, EMB)])


_gather_call = functools.partial(
    pl.kernel,
    out_type=(jax.ShapeDtypeStruct((BATCH, 128), jnp.float32),
              jax.ShapeDtypeStruct((BATCH, 128), jnp.float32),
              jax.ShapeDtypeStruct((BATCH, 128), jnp.float32)),
    mesh=plsc.VectorSubcoreMesh(core_axis_name="c", subcore_axis_name="s",
                                num_cores=NC, num_subcores=NS),
    scratch_types=[
        pltpu.VMEM((BPT,), jnp.int32),
        pltpu.VMEM((BPT,), jnp.int32),
        pltpu.VMEM((BPT, EMB), jnp.float32),
        pltpu.SemaphoreType.DMA,
    ],
    compiler_params=pltpu.CompilerParams(use_tc_tiling_on_sc=False),
)(_gather_body)

def _loss_body(u_ref, p_ref, n_ref, out_ref):
  u = u_ref[...]
  p = p_ref[...]
  n = n_ref[...]
  d = jnp.sum(u * p - u * n, axis=1)
  ls = jnp.minimum(d, 0.0) - jnp.log(1.0 + jnp.exp(-jnp.abs(d)))
  l2 = 0.5 * (jnp.sum(u * u) + jnp.sum(p * p) + jnp.sum(n * n))
  out_ref[0, 0] = -(jnp.sum(ls) / BATCH) + REG * (l2 / BATCH)


_loss_call = pl.pallas_call(
    _loss_body,
    grid=(1,),
    in_specs=[
        pl.BlockSpec((BATCH, 128), lambda i: (0, 0)),
        pl.BlockSpec((BATCH, 128), lambda i: (0, 0)),
        pl.BlockSpec((BATCH, 128), lambda i: (0, 0)),
    ],
    out_specs=pl.BlockSpec((1, 1), lambda i: (0, 0),
                           memory_space=pltpu.SMEM),
    out_shape=jax.ShapeDtypeStruct((1, 1), jnp.float32),
)


def kernel(u, i, j, edge_rows, edge_cols, edge_vals, user_embedding,
           item_embedding,
           W_one_0, b_one_0, W_two_0, b_two_0,
           W_one_1, b_one_1, W_two_1, b_two_1,
           W_one_2, b_one_2, W_two_2, b_two_2):
  f32 = jnp.float32
  eye8 = jnp.eye(8, dtype=f32)
  e_lo = jnp.concatenate([jnp.eye(HALF, dtype=f32),
                          jnp.zeros((HALF, HALF), f32)], axis=1)
  e_hi = jnp.concatenate([jnp.zeros((HALF, HALF), f32),
                          jnp.eye(HALF, dtype=f32)], axis=1)
  plo = jnp.kron(eye8, e_lo)
  phi = jnp.kron(eye8, e_hi)
  ones_bk = jnp.kron(eye8, jnp.ones((EMB, 1), f32))
  exp_bk = jnp.kron(eye8, jnp.ones((1, EMB), f32))

  ego_flat = jnp.concatenate(
      [user_embedding, item_embedding,
       jnp.zeros((NNP - NN, EMB), f32)], axis=0).reshape(NNP * EMB // 256, 256)

  rows2 = edge_rows.reshape(E // CHUNK, CHUNK)
  cols_lo = edge_cols * 2
  cols_hi = cols_lo + 1

  tables = []
  layer_ws = [(W_one_0, b_one_0, W_two_0, b_two_0),
              (W_one_1, b_one_1, W_two_1, b_two_1),
              (W_one_2, b_one_2, W_two_2, b_two_2)]
  for (w1, b1, w2, b2) in layer_ws:
    ego2 = ego_flat.reshape(NC * NNP, HALF)
    side3 = _spmm_call(ego2, cols_lo, cols_hi, rows2, edge_vals)
    side8 = side3.reshape(NC, NP8, 128)
    w1b = jnp.kron(eye8, w1)
    w2b = jnp.kron(eye8, w2)
    b1b = jnp.tile(b1, (1, 8))
    b2b = jnp.tile(b2, (1, 8))
    ego8, tnorm = _dense_call(side8, ego_flat.reshape(NP8, 256),
                              w1b, b1b, w2b, b2b, plo, phi, ones_bk, exp_bk)
    ego_flat = ego8
    tables.append(tnorm.reshape(NNP, EMB))

  uidx = u
  pidx = i + N_USERS
  nidx = j + N_USERS
  U, P, N = _gather_call(user_embedding, item_embedding,
                         tables[0], tables[1], tables[2],
                         uidx, pidx, nidx, i, j)
  loss = _loss_call(U, P, N)
  return loss[0, 0]


# speedup vs baseline: 1.0415x; 1.0415x over previous
import functools

import jax
import jax.numpy as jnp
from jax import lax
from jax.experimental import pallas as pl
from jax.experimental.pallas import tpu as pltpu
from jax.experimental.pallas import tpu_sc as plsc

N_USERS = 50000
N_ITEMS = 50000
NN = N_USERS + N_ITEMS
EMB = 32
HALF = 16
E = 1600000
BATCH = 4096
REG = 1e-05

NC = 2
NS = 16

BE = 512
CHUNK = 128
NCH = BE // CHUNK
NBLK = 196

NNP = 100352
ROWS_PER_TILE = NNP // NS
ZCH = 784


def _spmm_body(ego_ref, colslo_ref, colshi_ref, rows2_ref, vals_ref,
               out_ref,
               idx_0, vals_0, rows_0, idx_1, vals_1, rows_1,
               idx_2, vals_2, rows_2, idx_3, vals_3, rows_3,
               gbuf_a, gbuf_b, acc, *sems):
  stsem = sems[:4]
  gsem = sems[4:12]
  ssem = sems[12:20]
  c = lax.axis_index("c")
  s = lax.axis_index("s")
  sets = ((idx_0, vals_0, rows_0), (idx_1, vals_1, rows_1),
          (idx_2, vals_2, rows_2), (idx_3, vals_3, rows_3))
  gbufs = (gbuf_a, gbuf_b)

  @pl.loop(0, ZCH)
  def _(r):
    gbuf_a[r] = jnp.zeros((HALF,), jnp.float32)

  for t in range(ROWS_PER_TILE // ZCH):
    pltpu.sync_copy(gbuf_a.at[pl.ds(0, ZCH)],
                    acc.at[pl.ds(s * ROWS_PER_TILE + t * ZCH, ZCH)])
  plsc.subcore_barrier()

  def blk_off(b):
    fake = jnp.logical_and(b == NBLK - 1, s >= 5)
    return jnp.where(fake, 0, (s + 16 * b) * BE), fake

  def stage_fire(b, m):
    st_idx, st_vals, st_rows = sets[m]
    off, _ = blk_off(b)

    @pl.when(c == 0)
    def _():
      pltpu.async_copy(colslo_ref.at[pl.ds(off, BE)], st_idx, stsem[m])

    @pl.when(c == 1)
    def _():
      pltpu.async_copy(colshi_ref.at[pl.ds(off, BE)], st_idx, stsem[m])

    pltpu.async_copy(vals_ref.at[pl.ds(off, BE)], st_vals, stsem[m])
    pltpu.async_copy(rows2_ref.at[pl.ds(off // CHUNK, NCH), :], st_rows,
                     stsem[m])

  def stage_wait(m):
    st_idx, st_vals, st_rows = sets[m]
    pltpu.make_async_copy(colslo_ref.at[pl.ds(0, BE)], st_idx,
                          stsem[m]).wait()
    pltpu.make_async_copy(vals_ref.at[pl.ds(0, BE)], st_vals,
                          stsem[m]).wait()
    pltpu.make_async_copy(rows2_ref.at[pl.ds(0, NCH), :], st_rows,
                          stsem[m]).wait()

  def fire_gathers(m, p):
    st_idx = sets[m][0]
    gb = gbufs[p]
    for j in range(NCH):
      pltpu.async_copy(ego_ref.at[st_idx.at[pl.ds(j * CHUNK, CHUNK)]],
                       gb.at[pl.ds(j * CHUNK, CHUNK)], gsem[p * NCH + j])

  def scale_chunk(gb, j, st_vals):
    @plsc.parallel_loop(0, CHUNK // 16, unroll=4)
    def _(g):
      base_r = j * CHUNK + g * 16
      vv = st_vals[pl.ds(base_r, 16)]
      for k in range(16):
        r = base_r + k
        gb[r] = gb[r] * vv[k]

  def drain_scatters(p, mrows):
    rws = sets[mrows][2]
    gb = gbufs[p]
    for j in range(NCH):
      pltpu.make_async_copy(gb.at[pl.ds(j * CHUNK, CHUNK)],
                            acc.at[rws.at[j]], ssem[p * NCH + j]).wait()

  def proc(t, k):
    b = 4 * t + k
    p = k % 2
    q = 1 - p
    gb = gbufs[p]
    mc = k % 4
    mn = (k + 1) % 4
    mr = (k + 3) % 4
    _, fake = blk_off(b)

    if k == 0:
      @pl.when(t > 0)
      def _():
        drain_scatters(q, mr)
    else:
      drain_scatters(q, mr)

    if k == 0:
      stage_fire(b + 3, mr)
    else:
      @pl.when(t < NBLK // 4 - 1)
      def _():
        stage_fire(b + 3, mr)

    def prefetch():
      stage_wait(mn)
      fire_gathers(mn, q)

    if k == 3:
      @pl.when(t < NBLK // 4 - 1)
      def _():
        prefetch()
    else:
      prefetch()

    @pl.when(fake)
    def _():
      @pl.loop(0, BE // 16)
      def _(z):
        sets[mc][1][pl.ds(z * 16, 16)] = jnp.zeros((16,), jnp.float32)

    for j in range(NCH):
      pltpu.make_async_copy(ego_ref.at[sets[mc][0].at[pl.ds(j * CHUNK,
                                                            CHUNK)]],
                            gb.at[pl.ds(j * CHUNK, CHUNK)],
                            gsem[p * NCH + j]).wait()
      scale_chunk(gb, j, sets[mc][1])
      pltpu.async_copy(gb.at[pl.ds(j * CHUNK, CHUNK)],
                       acc.at[sets[mc][2].at[j]], ssem[p * NCH + j],
                       add=True)

  stage_fire(0, 0)
  stage_fire(1, 1)
  stage_fire(2, 2)
  stage_wait(0)
  fire_gathers(0, 0)

  @pl.loop(0, NBLK // 4)
  def _(t):
    for k in range(4):
      proc(t, k)

  drain_scatters((NBLK - 1) % 2, (NBLK - 1) % 4)

  plsc.subcore_barrier()

  for t in range(ROWS_PER_TILE // ZCH):
    r0 = s * ROWS_PER_TILE + t * ZCH
    pltpu.sync_copy(acc.at[pl.ds(r0, ZCH)], gbuf_a.at[pl.ds(0, ZCH)])
    pltpu.sync_copy(gbuf_a.at[pl.ds(0, ZCH)], out_ref.at[c, pl.ds(r0, ZCH), :])


_spmm_call = functools.partial(
    pl.kernel,
    out_type=jax.ShapeDtypeStruct((NC, NNP, HALF), jnp.float32),
    mesh=plsc.VectorSubcoreMesh(core_axis_name="c", subcore_axis_name="s",
                                num_cores=NC, num_subcores=NS),
    scratch_types=[
        pltpu.VMEM((BE,), jnp.int32),
        pltpu.VMEM((BE,), jnp.float32),
        pltpu.VMEM((NCH, CHUNK), jnp.int32),
    ] * 4 + [
        pltpu.VMEM((BE, HALF), jnp.float32),
        pltpu.VMEM((BE, HALF), jnp.float32),
        pltpu.VMEM_SHARED((NNP, HALF), jnp.float32),
    ] + [pltpu.SemaphoreType.DMA] * 20,
    compiler_params=pltpu.CompilerParams(use_tc_tiling_on_sc=False),
)(_spmm_body)


def _dense_body(lohi_ref, ego_ref, w1_ref, b1_ref, w2_ref, b2_ref,
                plo_ref, phi_ref, ones_ref, exp_ref, e_ref, tnorm_ref):
  side = (jnp.dot(lohi_ref[0], plo_ref[...],
                  preferred_element_type=jnp.float32) +
          jnp.dot(lohi_ref[1], phi_ref[...],
                  preferred_element_type=jnp.float32))
  ego = ego_ref[...]
  side_li = side + ego
  simple = jnp.dot(side_li, w1_ref[...], preferred_element_type=jnp.float32)
  inter = jnp.dot(ego * side, w2_ref[...], preferred_element_type=jnp.float32)
  e = simple + inter + b1_ref[...] + b2_ref[...]
  e = jnp.where(e >= 0, e, 0.01 * e)
  e_ref[...] = e
  nrm2 = jnp.dot(e * e, ones_ref[...], preferred_element_type=jnp.float32)
  scale = 1.0 / jnp.maximum(jnp.sqrt(nrm2), 1e-12)
  sw = jnp.dot(scale, exp_ref[...], preferred_element_type=jnp.float32)
  tnorm_ref[...] = e * sw


NP8 = NNP // 8
RB8 = NP8 // 4

_dense_call = pl.pallas_call(
    _dense_body,
    grid=(NP8 // RB8,),
    in_specs=[
        pl.BlockSpec((NC, RB8, 128), lambda i: (0, i, 0)),
        pl.BlockSpec((RB8, 256), lambda i: (i, 0)),
        pl.BlockSpec((256, 256), lambda i: (0, 0)),
        pl.BlockSpec((1, 256), lambda i: (0, 0)),
        pl.BlockSpec((256, 256), lambda i: (0, 0)),
        pl.BlockSpec((1, 256), lambda i: (0, 0)),
        pl.BlockSpec((128, 256), lambda i: (0, 0)),
        pl.BlockSpec((128, 256), lambda i: (0, 0)),
        pl.BlockSpec((256, 8), lambda i: (0, 0)),
        pl.BlockSpec((8, 256), lambda i: (0, 0)),
    ],
    out_specs=[
        pl.BlockSpec((RB8, 256), lambda i: (i, 0)),
        pl.BlockSpec((RB8, 256), lambda i: (i, 0)),
    ],
    out_shape=[
        jax.ShapeDtypeStruct((NP8, 256), jnp.float32),
        jax.ShapeDtypeStruct((NP8, 256), jnp.float32),
    ],
)

BPT = BATCH // (NC * NS)


def _gather_body(t0_ref, t1_ref, t2_ref, t3_ref, ui_ref, pi_ref, ni_ref,
                 uo_ref, po_ref, no_ref, iu, buf, sem):
  c = lax.axis_index("c")
  s = lax.axis_index("s")
  w = s * NC + c
  base = w * BPT
  tables = [t0_ref, t1_ref, t2_ref, t3_ref]
  for idx_ref, out_ref in ((ui_ref, uo_ref), (pi_ref, po_ref),
                           (ni_ref, no_ref)):
    pltpu.sync_copy(idx_ref.at[pl.ds(base, BPT)], iu)
    for k in range(4):
      pltpu.async_copy(tables[k].at[iu], buf, sem).wait()
      pltpu.sync_copy(buf, out_ref.at[pl.ds(base, BPT),
                                      pl.ds(k * EMB, EMB)])


_gather_call = functools.partial(
    pl.kernel,
    out_type=(jax.ShapeDtypeStruct((BATCH, 128), jnp.float32),
              jax.ShapeDtypeStruct((BATCH, 128), jnp.float32),
              jax.ShapeDtypeStruct((BATCH, 128), jnp.float32)),
    mesh=plsc.VectorSubcoreMesh(core_axis_name="c", subcore_axis_name="s",
                                num_cores=NC, num_subcores=NS),
    scratch_types=[
        pltpu.VMEM((BPT,), jnp.int32),
        pltpu.VMEM((BPT, EMB), jnp.float32),
        pltpu.SemaphoreType.DMA,
    ],
    compiler_params=pltpu.CompilerParams(use_tc_tiling_on_sc=False),
)(_gather_body)

def _loss_body(u_ref, p_ref, n_ref, out_ref):
  u = u_ref[...]
  p = p_ref[...]
  n = n_ref[...]
  d = jnp.sum(u * p - u * n, axis=1)
  ls = jnp.minimum(d, 0.0) - jnp.log(1.0 + jnp.exp(-jnp.abs(d)))
  l2 = 0.5 * (jnp.sum(u * u) + jnp.sum(p * p) + jnp.sum(n * n))
  out_ref[0, 0] = -(jnp.sum(ls) / BATCH) + REG * (l2 / BATCH)


_loss_call = pl.pallas_call(
    _loss_body,
    grid=(1,),
    in_specs=[
        pl.BlockSpec((BATCH, 128), lambda i: (0, 0)),
        pl.BlockSpec((BATCH, 128), lambda i: (0, 0)),
        pl.BlockSpec((BATCH, 128), lambda i: (0, 0)),
    ],
    out_specs=pl.BlockSpec((1, 1), lambda i: (0, 0),
                           memory_space=pltpu.SMEM),
    out_shape=jax.ShapeDtypeStruct((1, 1), jnp.float32),
)


def kernel(u, i, j, edge_rows, edge_cols, edge_vals, user_embedding,
           item_embedding,
           W_one_0, b_one_0, W_two_0, b_two_0,
           W_one_1, b_one_1, W_two_1, b_two_1,
           W_one_2, b_one_2, W_two_2, b_two_2):
  f32 = jnp.float32
  eye8 = jnp.eye(8, dtype=f32)
  e_lo = jnp.concatenate([jnp.eye(HALF, dtype=f32),
                          jnp.zeros((HALF, HALF), f32)], axis=1)
  e_hi = jnp.concatenate([jnp.zeros((HALF, HALF), f32),
                          jnp.eye(HALF, dtype=f32)], axis=1)
  plo = jnp.kron(eye8, e_lo)
  phi = jnp.kron(eye8, e_hi)
  ones_bk = jnp.kron(eye8, jnp.ones((EMB, 1), f32))
  exp_bk = jnp.kron(eye8, jnp.ones((1, EMB), f32))

  ego_flat = jnp.concatenate(
      [user_embedding, item_embedding,
       jnp.zeros((NNP - NN, EMB), f32)], axis=0).reshape(NNP * EMB // 256, 256)

  rows2 = edge_rows.reshape(E // CHUNK, CHUNK)
  cols_lo = edge_cols * 2
  cols_hi = cols_lo + 1

  tables = [ego_flat.reshape(NNP, EMB)]
  layer_ws = [(W_one_0, b_one_0, W_two_0, b_two_0),
              (W_one_1, b_one_1, W_two_1, b_two_1),
              (W_one_2, b_one_2, W_two_2, b_two_2)]
  for (w1, b1, w2, b2) in layer_ws:
    ego2 = ego_flat.reshape(NC * NNP, HALF)
    side3 = _spmm_call(ego2, cols_lo, cols_hi, rows2, edge_vals)
    side8 = side3.reshape(NC, NP8, 128)
    w1b = jnp.kron(eye8, w1)
    w2b = jnp.kron(eye8, w2)
    b1b = jnp.tile(b1, (1, 8))
    b2b = jnp.tile(b2, (1, 8))
    ego8, tnorm = _dense_call(side8, ego_flat.reshape(NP8, 256),
                              w1b, b1b, w2b, b2b, plo, phi, ones_bk, exp_bk)
    ego_flat = ego8
    tables.append(tnorm.reshape(NNP, EMB))

  uidx = u
  pidx = i + N_USERS
  nidx = j + N_USERS
  U, P, N = _gather_call(tables[0], tables[1], tables[2], tables[3],
                         uidx, pidx, nidx)
  loss = _loss_call(U, P, N)
  return loss[0, 0]


# speedup vs baseline: 1.1328x; 1.0877x over previous
import functools

import jax
import jax.numpy as jnp
from jax import lax
from jax.experimental import pallas as pl
from jax.experimental.pallas import tpu as pltpu
from jax.experimental.pallas import tpu_sc as plsc

N_USERS = 50000
N_ITEMS = 50000
NN = N_USERS + N_ITEMS
EMB = 32
HALF = 16
E = 1600000
BATCH = 4096
REG = 1e-05

NC = 2
NS = 16

BE = 512
CHUNK = 128
NCH = BE // CHUNK
NBLK = 196

NNP = 100352
ROWS_PER_TILE = NNP // NS
ZCH = 784


def _spmm_body(ego_ref, colslo_ref, colshi_ref, rows2_ref, vals_ref,
               out_ref,
               idx_0, vals_0, rows_0, idx_1, vals_1, rows_1,
               idx_2, vals_2, rows_2, idx_3, vals_3, rows_3,
               gbuf_a, gbuf_b, acc, *sems):
  stsem = sems[:4]
  gsem = sems[4:12]
  ssem = sems[12:20]
  c = lax.axis_index("c")
  s = lax.axis_index("s")
  sets = ((idx_0, vals_0, rows_0), (idx_1, vals_1, rows_1),
          (idx_2, vals_2, rows_2), (idx_3, vals_3, rows_3))
  gbufs = (gbuf_a, gbuf_b)

  @pl.loop(0, ZCH)
  def _(r):
    gbuf_a[r] = jnp.zeros((HALF,), jnp.float32)

  for t in range(ROWS_PER_TILE // ZCH):
    pltpu.sync_copy(gbuf_a.at[pl.ds(0, ZCH)],
                    acc.at[pl.ds(s * ROWS_PER_TILE + t * ZCH, ZCH)])
  plsc.subcore_barrier()

  def blk_off(b):
    fake = jnp.logical_and(b == NBLK - 1, s >= 5)
    return jnp.where(fake, 0, (s + 16 * b) * BE), fake

  def stage_fire(b, m):
    st_idx, st_vals, st_rows = sets[m]
    off, _ = blk_off(b)

    @pl.when(c == 0)
    def _():
      pltpu.async_copy(colslo_ref.at[pl.ds(off, BE)], st_idx, stsem[m])

    @pl.when(c == 1)
    def _():
      pltpu.async_copy(colshi_ref.at[pl.ds(off, BE)], st_idx, stsem[m])

    pltpu.async_copy(vals_ref.at[pl.ds(off, BE)], st_vals, stsem[m])
    pltpu.async_copy(rows2_ref.at[pl.ds(off // CHUNK, NCH), :], st_rows,
                     stsem[m])

  def stage_wait(m):
    st_idx, st_vals, st_rows = sets[m]
    pltpu.make_async_copy(colslo_ref.at[pl.ds(0, BE)], st_idx,
                          stsem[m]).wait()
    pltpu.make_async_copy(vals_ref.at[pl.ds(0, BE)], st_vals,
                          stsem[m]).wait()
    pltpu.make_async_copy(rows2_ref.at[pl.ds(0, NCH), :], st_rows,
                          stsem[m]).wait()

  def fire_gathers(m, p):
    st_idx = sets[m][0]
    gb = gbufs[p]
    for j in range(NCH):
      pltpu.async_copy(ego_ref.at[st_idx.at[pl.ds(j * CHUNK, CHUNK)]],
                       gb.at[pl.ds(j * CHUNK, CHUNK)], gsem[p * NCH + j])

  def scale_chunk(gb, j, st_vals):
    @plsc.parallel_loop(0, CHUNK // 16, unroll=4)
    def _(g):
      base_r = j * CHUNK + g * 16
      vv = st_vals[pl.ds(base_r, 16)]
      for k in range(16):
        r = base_r + k
        gb[r] = gb[r] * vv[k]

  def drain_scatters(p, mrows):
    rws = sets[mrows][2]
    gb = gbufs[p]
    for j in range(NCH):
      pltpu.make_async_copy(gb.at[pl.ds(j * CHUNK, CHUNK)],
                            acc.at[rws.at[j]], ssem[p * NCH + j]).wait()

  def proc(t, k):
    b = 4 * t + k
    p = k % 2
    q = 1 - p
    gb = gbufs[p]
    mc = k % 4
    mn = (k + 1) % 4
    mr = (k + 3) % 4
    _, fake = blk_off(b)

    if k == 0:
      @pl.when(t > 0)
      def _():
        drain_scatters(q, mr)
    else:
      drain_scatters(q, mr)

    if k == 0:
      stage_fire(b + 3, mr)
    else:
      @pl.when(t < NBLK // 4 - 1)
      def _():
        stage_fire(b + 3, mr)

    def prefetch():
      stage_wait(mn)
      fire_gathers(mn, q)

    if k == 3:
      @pl.when(t < NBLK // 4 - 1)
      def _():
        prefetch()
    else:
      prefetch()

    @pl.when(fake)
    def _():
      @pl.loop(0, BE // 16)
      def _(z):
        sets[mc][1][pl.ds(z * 16, 16)] = jnp.zeros((16,), jnp.float32)

    for j in range(NCH):
      pltpu.make_async_copy(ego_ref.at[sets[mc][0].at[pl.ds(j * CHUNK,
                                                            CHUNK)]],
                            gb.at[pl.ds(j * CHUNK, CHUNK)],
                            gsem[p * NCH + j]).wait()
      scale_chunk(gb, j, sets[mc][1])
      pltpu.async_copy(gb.at[pl.ds(j * CHUNK, CHUNK)],
                       acc.at[sets[mc][2].at[j]], ssem[p * NCH + j],
                       add=True)

  stage_fire(0, 0)
  stage_fire(1, 1)
  stage_fire(2, 2)
  stage_wait(0)
  fire_gathers(0, 0)

  @pl.loop(0, NBLK // 4)
  def _(t):
    for k in range(4):
      proc(t, k)

  drain_scatters((NBLK - 1) % 2, (NBLK - 1) % 4)

  plsc.subcore_barrier()

  for t in range(ROWS_PER_TILE // ZCH):
    r0 = s * ROWS_PER_TILE + t * ZCH
    pltpu.sync_copy(acc.at[pl.ds(r0, ZCH)], gbuf_a.at[pl.ds(0, ZCH)])
    pltpu.sync_copy(gbuf_a.at[pl.ds(0, ZCH)], out_ref.at[c, pl.ds(r0, ZCH), :])


_spmm_call = functools.partial(
    pl.kernel,
    out_type=jax.ShapeDtypeStruct((NC, NNP, HALF), jnp.float32),
    mesh=plsc.VectorSubcoreMesh(core_axis_name="c", subcore_axis_name="s",
                                num_cores=NC, num_subcores=NS),
    scratch_types=[
        pltpu.VMEM((BE,), jnp.int32),
        pltpu.VMEM((BE,), jnp.float32),
        pltpu.VMEM((NCH, CHUNK), jnp.int32),
    ] * 4 + [
        pltpu.VMEM((BE, HALF), jnp.float32),
        pltpu.VMEM((BE, HALF), jnp.float32),
        pltpu.VMEM_SHARED((NNP, HALF), jnp.float32),
    ] + [pltpu.SemaphoreType.DMA] * 20,
    compiler_params=pltpu.CompilerParams(use_tc_tiling_on_sc=False),
)(_spmm_body)


def _dense_body(lohi_ref, ego_ref, w1_ref, b1_ref, w2_ref, b2_ref,
                plo_ref, phi_ref, ones_ref, exp_ref, e_ref, tnorm_ref):
  side = (jnp.dot(lohi_ref[0], plo_ref[...],
                  preferred_element_type=jnp.float32) +
          jnp.dot(lohi_ref[1], phi_ref[...],
                  preferred_element_type=jnp.float32))
  ego = ego_ref[...].reshape(RB8, 256)
  side_li = side + ego
  simple = jnp.dot(side_li, w1_ref[...], preferred_element_type=jnp.float32)
  inter = jnp.dot(ego * side, w2_ref[...], preferred_element_type=jnp.float32)
  e = simple + inter + b1_ref[...] + b2_ref[...]
  e = jnp.where(e >= 0, e, 0.01 * e)
  e_ref[...] = e.reshape(2 * RB8, 128)
  nrm2 = jnp.dot(e * e, ones_ref[...], preferred_element_type=jnp.float32)
  scale = 1.0 / jnp.maximum(jnp.sqrt(nrm2), 1e-12)
  sw = jnp.dot(scale, exp_ref[...], preferred_element_type=jnp.float32)
  tnorm_ref[...] = (e * sw).reshape(2 * RB8, 128)


NP8 = NNP // 8
RB8 = NP8 // 4

_dense_call = pl.pallas_call(
    _dense_body,
    grid=(NP8 // RB8,),
    in_specs=[
        pl.BlockSpec((NC, RB8, 128), lambda i: (0, i, 0)),
        pl.BlockSpec((2 * RB8, 128), lambda i: (i, 0)),
        pl.BlockSpec((256, 256), lambda i: (0, 0)),
        pl.BlockSpec((1, 256), lambda i: (0, 0)),
        pl.BlockSpec((256, 256), lambda i: (0, 0)),
        pl.BlockSpec((1, 256), lambda i: (0, 0)),
        pl.BlockSpec((128, 256), lambda i: (0, 0)),
        pl.BlockSpec((128, 256), lambda i: (0, 0)),
        pl.BlockSpec((256, 8), lambda i: (0, 0)),
        pl.BlockSpec((8, 256), lambda i: (0, 0)),
    ],
    out_specs=[
        pl.BlockSpec((2 * RB8, 128), lambda i: (i, 0)),
        pl.BlockSpec((2 * RB8, 128), lambda i: (i, 0)),
    ],
    out_shape=[
        jax.ShapeDtypeStruct((2 * NP8, 128), jnp.float32),
        jax.ShapeDtypeStruct((2 * NP8, 128), jnp.float32),
    ],
)

BPT = BATCH // (NC * NS)


def _gather_body(t0_ref, t1_ref, t2_ref, t3_ref, ui_ref, pi_ref, ni_ref,
                 uo_ref, po_ref, no_ref, iu, buf, sem):
  c = lax.axis_index("c")
  s = lax.axis_index("s")
  w = s * NC + c
  base = w * BPT
  tables = [t0_ref, t1_ref, t2_ref, t3_ref]
  for idx_ref, out_ref in ((ui_ref, uo_ref), (pi_ref, po_ref),
                           (ni_ref, no_ref)):
    pltpu.sync_copy(idx_ref.at[pl.ds(base, BPT)], iu)
    for k in range(4):
      pltpu.async_copy(tables[k].at[iu], buf, sem).wait()
      pltpu.sync_copy(buf, out_ref.at[pl.ds(base, BPT),
                                      pl.ds(k * EMB, EMB)])


_gather_call = functools.partial(
    pl.kernel,
    out_type=(jax.ShapeDtypeStruct((BATCH, 128), jnp.float32),
              jax.ShapeDtypeStruct((BATCH, 128), jnp.float32),
              jax.ShapeDtypeStruct((BATCH, 128), jnp.float32)),
    mesh=plsc.VectorSubcoreMesh(core_axis_name="c", subcore_axis_name="s",
                                num_cores=NC, num_subcores=NS),
    scratch_types=[
        pltpu.VMEM((BPT,), jnp.int32),
        pltpu.VMEM((BPT, EMB), jnp.float32),
        pltpu.SemaphoreType.DMA,
    ],
    compiler_params=pltpu.CompilerParams(use_tc_tiling_on_sc=False),
)(_gather_body)

def _loss_body(u_ref, p_ref, n_ref, out_ref):
  u = u_ref[...]
  p = p_ref[...]
  n = n_ref[...]
  d = jnp.sum(u * p - u * n, axis=1)
  ls = jnp.minimum(d, 0.0) - jnp.log(1.0 + jnp.exp(-jnp.abs(d)))
  l2 = 0.5 * (jnp.sum(u * u) + jnp.sum(p * p) + jnp.sum(n * n))
  out_ref[0, 0] = -(jnp.sum(ls) / BATCH) + REG * (l2 / BATCH)


_loss_call = pl.pallas_call(
    _loss_body,
    grid=(1,),
    in_specs=[
        pl.BlockSpec((BATCH, 128), lambda i: (0, 0)),
        pl.BlockSpec((BATCH, 128), lambda i: (0, 0)),
        pl.BlockSpec((BATCH, 128), lambda i: (0, 0)),
    ],
    out_specs=pl.BlockSpec((1, 1), lambda i: (0, 0),
                           memory_space=pltpu.SMEM),
    out_shape=jax.ShapeDtypeStruct((1, 1), jnp.float32),
)


def kernel(u, i, j, edge_rows, edge_cols, edge_vals, user_embedding,
           item_embedding,
           W_one_0, b_one_0, W_two_0, b_two_0,
           W_one_1, b_one_1, W_two_1, b_two_1,
           W_one_2, b_one_2, W_two_2, b_two_2):
  f32 = jnp.float32
  eye8 = jnp.eye(8, dtype=f32)
  e_lo = jnp.concatenate([jnp.eye(HALF, dtype=f32),
                          jnp.zeros((HALF, HALF), f32)], axis=1)
  e_hi = jnp.concatenate([jnp.zeros((HALF, HALF), f32),
                          jnp.eye(HALF, dtype=f32)], axis=1)
  plo = jnp.kron(eye8, e_lo)
  phi = jnp.kron(eye8, e_hi)
  ones_bk = jnp.kron(eye8, jnp.ones((EMB, 1), f32))
  exp_bk = jnp.kron(eye8, jnp.ones((1, EMB), f32))

  ego_flat = jnp.concatenate(
      [user_embedding, item_embedding,
       jnp.zeros((NNP - NN, EMB), f32)], axis=0).reshape(NNP * EMB // 128, 128)

  rows2 = edge_rows.reshape(E // CHUNK, CHUNK)
  cols_lo = edge_cols * 2
  cols_hi = cols_lo + 1

  tables = [ego_flat.reshape(NNP, EMB)]
  layer_ws = [(W_one_0, b_one_0, W_two_0, b_two_0),
              (W_one_1, b_one_1, W_two_1, b_two_1),
              (W_one_2, b_one_2, W_two_2, b_two_2)]
  for (w1, b1, w2, b2) in layer_ws:
    ego2 = ego_flat.reshape(NC * NNP, HALF)
    side3 = _spmm_call(ego2, cols_lo, cols_hi, rows2, edge_vals)
    side8 = side3.reshape(NC, NP8, 128)
    w1b = jnp.kron(eye8, w1)
    w2b = jnp.kron(eye8, w2)
    b1b = jnp.tile(b1, (1, 8))
    b2b = jnp.tile(b2, (1, 8))
    ego8, tnorm = _dense_call(side8, ego_flat,
                              w1b, b1b, w2b, b2b, plo, phi, ones_bk, exp_bk)
    ego_flat = ego8
    tables.append(tnorm.reshape(NNP, EMB))

  uidx = u
  pidx = i + N_USERS
  nidx = j + N_USERS
  U, P, N = _gather_call(tables[0], tables[1], tables[2], tables[3],
                         uidx, pidx, nidx)
  loss = _loss_call(U, P, N)
  return loss[0, 0]


# speedup vs baseline: 1.1456x; 1.0113x over previous
import functools

import jax
import jax.numpy as jnp
from jax import lax
from jax.experimental import pallas as pl
from jax.experimental.pallas import tpu as pltpu
from jax.experimental.pallas import tpu_sc as plsc

N_USERS = 50000
N_ITEMS = 50000
NN = N_USERS + N_ITEMS
EMB = 32
HALF = 16
E = 1600000
BATCH = 4096
REG = 1e-05

NC = 2
NS = 16

BE = 512
CHUNK = 128
NCH = BE // CHUNK
NBLK = 196

NNP = 100352
ROWS_PER_TILE = NNP // NS
ZCH = 784


def _spmm_body(ego_ref, colslo_ref, colshi_ref, rows2_ref, vals_ref,
               out_ref,
               idx_0, vals_0, rows_0, idx_1, vals_1, rows_1,
               idx_2, vals_2, rows_2, idx_3, vals_3, rows_3,
               gbuf_a, gbuf_b, acc, *sems):
  stsem = sems[:4]
  gsem = sems[4:12]
  ssem = sems[12:20]
  c = lax.axis_index("c")
  s = lax.axis_index("s")
  sets = ((idx_0, vals_0, rows_0), (idx_1, vals_1, rows_1),
          (idx_2, vals_2, rows_2), (idx_3, vals_3, rows_3))
  gbufs = (gbuf_a, gbuf_b)

  @pl.loop(0, ZCH)
  def _(r):
    gbuf_a[r] = jnp.zeros((HALF,), jnp.float32)

  zds = [
      pltpu.async_copy(gbuf_a.at[pl.ds(0, ZCH)],
                       acc.at[pl.ds(s * ROWS_PER_TILE + t * ZCH, ZCH)],
                       sems[4])
      for t in range(ROWS_PER_TILE // ZCH)
  ]
  for d in zds:
    d.wait()
  plsc.subcore_barrier()

  def blk_off(b):
    fake = jnp.logical_and(b == NBLK - 1, s >= 5)
    return jnp.where(fake, 0, (s + 16 * b) * BE), fake

  def stage_fire(b, m):
    st_idx, st_vals, st_rows = sets[m]
    off, _ = blk_off(b)

    @pl.when(c == 0)
    def _():
      pltpu.async_copy(colslo_ref.at[pl.ds(off, BE)], st_idx, stsem[m])

    @pl.when(c == 1)
    def _():
      pltpu.async_copy(colshi_ref.at[pl.ds(off, BE)], st_idx, stsem[m])

    pltpu.async_copy(vals_ref.at[pl.ds(off, BE)], st_vals, stsem[m])
    pltpu.async_copy(rows2_ref.at[pl.ds(off // CHUNK, NCH), :], st_rows,
                     stsem[m])

  def stage_wait(m):
    st_idx, st_vals, st_rows = sets[m]
    pltpu.make_async_copy(colslo_ref.at[pl.ds(0, BE)], st_idx,
                          stsem[m]).wait()
    pltpu.make_async_copy(vals_ref.at[pl.ds(0, BE)], st_vals,
                          stsem[m]).wait()
    pltpu.make_async_copy(rows2_ref.at[pl.ds(0, NCH), :], st_rows,
                          stsem[m]).wait()

  def fire_gathers(m, p):
    st_idx = sets[m][0]
    gb = gbufs[p]
    for j in range(NCH):
      pltpu.async_copy(ego_ref.at[st_idx.at[pl.ds(j * CHUNK, CHUNK)]],
                       gb.at[pl.ds(j * CHUNK, CHUNK)], gsem[p * NCH + j])

  def scale_chunk(gb, j, st_vals):
    @plsc.parallel_loop(0, CHUNK // 16, unroll=4)
    def _(g):
      base_r = j * CHUNK + g * 16
      vv = st_vals[pl.ds(base_r, 16)]
      for k in range(16):
        r = base_r + k
        gb[r] = gb[r] * vv[k]

  def drain_scatters(p, mrows):
    rws = sets[mrows][2]
    gb = gbufs[p]
    for j in range(NCH):
      pltpu.make_async_copy(gb.at[pl.ds(j * CHUNK, CHUNK)],
                            acc.at[rws.at[j]], ssem[p * NCH + j]).wait()

  def proc(t, k):
    b = 4 * t + k
    p = k % 2
    q = 1 - p
    gb = gbufs[p]
    mc = k % 4
    mn = (k + 1) % 4
    mr = (k + 3) % 4
    _, fake = blk_off(b)

    if k == 0:
      @pl.when(t > 0)
      def _():
        drain_scatters(q, mr)
    else:
      drain_scatters(q, mr)

    if k == 0:
      stage_fire(b + 3, mr)
    else:
      @pl.when(t < NBLK // 4 - 1)
      def _():
        stage_fire(b + 3, mr)

    def prefetch():
      stage_wait(mn)
      fire_gathers(mn, q)

    if k == 3:
      @pl.when(t < NBLK // 4 - 1)
      def _():
        prefetch()
    else:
      prefetch()

    @pl.when(fake)
    def _():
      @pl.loop(0, BE // 16)
      def _(z):
        sets[mc][1][pl.ds(z * 16, 16)] = jnp.zeros((16,), jnp.float32)

    for j in range(NCH):
      pltpu.make_async_copy(ego_ref.at[sets[mc][0].at[pl.ds(j * CHUNK,
                                                            CHUNK)]],
                            gb.at[pl.ds(j * CHUNK, CHUNK)],
                            gsem[p * NCH + j]).wait()
      scale_chunk(gb, j, sets[mc][1])
      pltpu.async_copy(gb.at[pl.ds(j * CHUNK, CHUNK)],
                       acc.at[sets[mc][2].at[j]], ssem[p * NCH + j],
                       add=True)

  stage_fire(0, 0)
  stage_fire(1, 1)
  stage_fire(2, 2)
  stage_wait(0)
  fire_gathers(0, 0)

  @pl.loop(0, NBLK // 4)
  def _(t):
    for k in range(4):
      proc(t, k)

  drain_scatters((NBLK - 1) % 2, (NBLK - 1) % 4)

  plsc.subcore_barrier()

  nco = ROWS_PER_TILE // ZCH
  for t in range(nco):
    r0 = s * ROWS_PER_TILE + t * ZCH
    gb = gbufs[t % 2]
    if t >= 2:
      rp = s * ROWS_PER_TILE + (t - 2) * ZCH
      pltpu.make_async_copy(gb.at[pl.ds(0, ZCH)],
                            out_ref.at[c, pl.ds(rp, ZCH), :],
                            sems[5 + t % 2]).wait()
    pltpu.sync_copy(acc.at[pl.ds(r0, ZCH)], gb.at[pl.ds(0, ZCH)])
    pltpu.async_copy(gb.at[pl.ds(0, ZCH)], out_ref.at[c, pl.ds(r0, ZCH), :],
                     sems[5 + t % 2])
  for t in range(nco - 2, nco):
    r0 = s * ROWS_PER_TILE + t * ZCH
    pltpu.make_async_copy(gbufs[t % 2].at[pl.ds(0, ZCH)],
                          out_ref.at[c, pl.ds(r0, ZCH), :],
                          sems[5 + t % 2]).wait()


_spmm_call = functools.partial(
    pl.kernel,
    out_type=jax.ShapeDtypeStruct((NC, NNP, HALF), jnp.float32),
    mesh=plsc.VectorSubcoreMesh(core_axis_name="c", subcore_axis_name="s",
                                num_cores=NC, num_subcores=NS),
    scratch_types=[
        pltpu.VMEM((BE,), jnp.int32),
        pltpu.VMEM((BE,), jnp.float32),
        pltpu.VMEM((NCH, CHUNK), jnp.int32),
    ] * 4 + [
        pltpu.VMEM((BE, HALF), jnp.float32),
        pltpu.VMEM((BE, HALF), jnp.float32),
        pltpu.VMEM_SHARED((NNP, HALF), jnp.float32),
    ] + [pltpu.SemaphoreType.DMA] * 20,
    compiler_params=pltpu.CompilerParams(use_tc_tiling_on_sc=False),
)(_spmm_body)


def _dense_body(lohi_ref, ego_ref, w1_ref, b1_ref, w2_ref, b2_ref,
                plo_ref, phi_ref, ones_ref, exp_ref, e_ref, tnorm_ref):
  side = (jnp.dot(lohi_ref[0], plo_ref[...],
                  preferred_element_type=jnp.float32) +
          jnp.dot(lohi_ref[1], phi_ref[...],
                  preferred_element_type=jnp.float32))
  ego = ego_ref[...].reshape(RB8, 256)
  side_li = side + ego
  simple = jnp.dot(side_li, w1_ref[...], preferred_element_type=jnp.float32)
  inter = jnp.dot(ego * side, w2_ref[...], preferred_element_type=jnp.float32)
  e = simple + inter + b1_ref[...] + b2_ref[...]
  e = jnp.where(e >= 0, e, 0.01 * e)
  e_ref[...] = e.reshape(2 * RB8, 128)
  nrm2 = jnp.dot(e * e, ones_ref[...], preferred_element_type=jnp.float32)
  scale = 1.0 / jnp.maximum(jnp.sqrt(nrm2), 1e-12)
  sw = jnp.dot(scale, exp_ref[...], preferred_element_type=jnp.float32)
  tnorm_ref[...] = (e * sw).reshape(2 * RB8, 128)


NP8 = NNP // 8
RB8 = NP8 // 4

_dense_call = pl.pallas_call(
    _dense_body,
    grid=(NP8 // RB8,),
    in_specs=[
        pl.BlockSpec((NC, RB8, 128), lambda i: (0, i, 0)),
        pl.BlockSpec((2 * RB8, 128), lambda i: (i, 0)),
        pl.BlockSpec((256, 256), lambda i: (0, 0)),
        pl.BlockSpec((1, 256), lambda i: (0, 0)),
        pl.BlockSpec((256, 256), lambda i: (0, 0)),
        pl.BlockSpec((1, 256), lambda i: (0, 0)),
        pl.BlockSpec((128, 256), lambda i: (0, 0)),
        pl.BlockSpec((128, 256), lambda i: (0, 0)),
        pl.BlockSpec((256, 8), lambda i: (0, 0)),
        pl.BlockSpec((8, 256), lambda i: (0, 0)),
    ],
    out_specs=[
        pl.BlockSpec((2 * RB8, 128), lambda i: (i, 0)),
        pl.BlockSpec((2 * RB8, 128), lambda i: (i, 0)),
    ],
    out_shape=[
        jax.ShapeDtypeStruct((2 * NP8, 128), jnp.float32),
        jax.ShapeDtypeStruct((2 * NP8, 128), jnp.float32),
    ],
)

BPT = BATCH // (NC * NS)


def _gather_body(t0_ref, t1_ref, t2_ref, t3_ref, ui_ref, pi_ref, ni_ref,
                 uo_ref, po_ref, no_ref, iu, buf, sem):
  c = lax.axis_index("c")
  s = lax.axis_index("s")
  w = s * NC + c
  base = w * BPT
  tables = [t0_ref, t1_ref, t2_ref, t3_ref]
  for idx_ref, out_ref in ((ui_ref, uo_ref), (pi_ref, po_ref),
                           (ni_ref, no_ref)):
    pltpu.sync_copy(idx_ref.at[pl.ds(base, BPT)], iu)
    for k in range(4):
      pltpu.async_copy(tables[k].at[iu], buf, sem).wait()
      pltpu.sync_copy(buf, out_ref.at[pl.ds(base, BPT),
                                      pl.ds(k * EMB, EMB)])


_gather_call = functools.partial(
    pl.kernel,
    out_type=(jax.ShapeDtypeStruct((BATCH, 128), jnp.float32),
              jax.ShapeDtypeStruct((BATCH, 128), jnp.float32),
              jax.ShapeDtypeStruct((BATCH, 128), jnp.float32)),
    mesh=plsc.VectorSubcoreMesh(core_axis_name="c", subcore_axis_name="s",
                                num_cores=NC, num_subcores=NS),
    scratch_types=[
        pltpu.VMEM((BPT,), jnp.int32),
        pltpu.VMEM((BPT, EMB), jnp.float32),
        pltpu.SemaphoreType.DMA,
    ],
    compiler_params=pltpu.CompilerParams(use_tc_tiling_on_sc=False),
)(_gather_body)

def _loss_body(u_ref, p_ref, n_ref, out_ref):
  u = u_ref[...]
  p = p_ref[...]
  n = n_ref[...]
  d = jnp.sum(u * p - u * n, axis=1)
  ls = jnp.minimum(d, 0.0) - jnp.log(1.0 + jnp.exp(-jnp.abs(d)))
  l2 = 0.5 * (jnp.sum(u * u) + jnp.sum(p * p) + jnp.sum(n * n))
  out_ref[0, 0] = -(jnp.sum(ls) / BATCH) + REG * (l2 / BATCH)


_loss_call = pl.pallas_call(
    _loss_body,
    grid=(1,),
    in_specs=[
        pl.BlockSpec((BATCH, 128), lambda i: (0, 0)),
        pl.BlockSpec((BATCH, 128), lambda i: (0, 0)),
        pl.BlockSpec((BATCH, 128), lambda i: (0, 0)),
    ],
    out_specs=pl.BlockSpec((1, 1), lambda i: (0, 0),
                           memory_space=pltpu.SMEM),
    out_shape=jax.ShapeDtypeStruct((1, 1), jnp.float32),
)


def kernel(u, i, j, edge_rows, edge_cols, edge_vals, user_embedding,
           item_embedding,
           W_one_0, b_one_0, W_two_0, b_two_0,
           W_one_1, b_one_1, W_two_1, b_two_1,
           W_one_2, b_one_2, W_two_2, b_two_2):
  f32 = jnp.float32
  eye8 = jnp.eye(8, dtype=f32)
  e_lo = jnp.concatenate([jnp.eye(HALF, dtype=f32),
                          jnp.zeros((HALF, HALF), f32)], axis=1)
  e_hi = jnp.concatenate([jnp.zeros((HALF, HALF), f32),
                          jnp.eye(HALF, dtype=f32)], axis=1)
  plo = jnp.kron(eye8, e_lo)
  phi = jnp.kron(eye8, e_hi)
  ones_bk = jnp.kron(eye8, jnp.ones((EMB, 1), f32))
  exp_bk = jnp.kron(eye8, jnp.ones((1, EMB), f32))

  ego_flat = jnp.concatenate(
      [user_embedding, item_embedding,
       jnp.zeros((NNP - NN, EMB), f32)], axis=0).reshape(NNP * EMB // 128, 128)

  rows2 = edge_rows.reshape(E // CHUNK, CHUNK)
  cols_lo = edge_cols * 2
  cols_hi = cols_lo + 1

  tables = [ego_flat.reshape(NNP, EMB)]
  layer_ws = [(W_one_0, b_one_0, W_two_0, b_two_0),
              (W_one_1, b_one_1, W_two_1, b_two_1),
              (W_one_2, b_one_2, W_two_2, b_two_2)]
  for (w1, b1, w2, b2) in layer_ws:
    ego2 = ego_flat.reshape(NC * NNP, HALF)
    side3 = _spmm_call(ego2, cols_lo, cols_hi, rows2, edge_vals)
    side8 = side3.reshape(NC, NP8, 128)
    w1b = jnp.kron(eye8, w1)
    w2b = jnp.kron(eye8, w2)
    b1b = jnp.tile(b1, (1, 8))
    b2b = jnp.tile(b2, (1, 8))
    ego8, tnorm = _dense_call(side8, ego_flat,
                              w1b, b1b, w2b, b2b, plo, phi, ones_bk, exp_bk)
    ego_flat = ego8
    tables.append(tnorm.reshape(NNP, EMB))

  uidx = u
  pidx = i + N_USERS
  nidx = j + N_USERS
  U, P, N = _gather_call(tables[0], tables[1], tables[2], tables[3],
                         uidx, pidx, nidx)
  loss = _loss_call(U, P, N)
  return loss[0, 0]
